# baseline (speedup 1.0000x reference)
import functools

import jax
import jax.numpy as jnp
from jax import lax
from jax.experimental import pallas as pl
from jax.experimental.pallas import tpu as pltpu
from jax.experimental.pallas import tpu_sc as plsc

H = 64
_EPS = 1e-5
_NW = 32
_CHUNK = 128


def _ln(o, g, bt):
    mu = jnp.mean(o, axis=-1, keepdims=True)
    v = jnp.mean((o - mu) ** 2, axis=-1, keepdims=True)
    return g * (o - mu) * lax.rsqrt(v + _EPS) + bt


def _wspec(a):
    nd = a.ndim
    return pl.BlockSpec(a.shape, lambda i, _nd=nd: (0,) * _nd)


def _mw(m):
    return (m['W1'], m['b1'].reshape(1, -1), m['W2'], m['b2'].reshape(1, -1),
            m['g'].reshape(1, -1), m['bt'].reshape(1, -1))


def _enc_body(x_ref, w1_ref, b1_ref, w2_ref, b2_ref, g_ref, bt_ref, o_ref):
    h = jnp.maximum(
        jnp.dot(x_ref[...], w1_ref[...], preferred_element_type=jnp.float32)
        + b1_ref[...], 0.0)
    o = jnp.dot(h, w2_ref[...], preferred_element_type=jnp.float32) + b2_ref[...]
    o_ref[...] = _ln(o, g_ref[...], bt_ref[...])


def _enc(x, m, row_block, interpret=False):
    n, f = x.shape
    ws = _mw(m)
    return pl.pallas_call(
        _enc_body,
        grid=(n // row_block,),
        in_specs=[pl.BlockSpec((row_block, f), lambda i: (i, 0))]
        + [_wspec(w) for w in ws],
        out_specs=pl.BlockSpec((row_block, H), lambda i: (i, 0)),
        out_shape=jax.ShapeDtypeStruct((n, H), jnp.float32),
        interpret=interpret,
    )(x, *ws)


def _gru_body(x_ref, wih_ref, bih_ref, whh_ref, bhh_ref,
              w1_ref, b1_ref, w2_ref, b2_ref, g_ref, bt_ref, o_ref):
    r_rows = x_ref.shape[1]
    wih = wih_ref[...]
    bih = bih_ref[...]
    whh = whh_ref[...]
    bhh = bhh_ref[...]
    h = jnp.zeros((r_rows, 16), jnp.float32)
    for t in range(8):
        xt = x_ref[t]
        gi = jnp.dot(xt, wih, preferred_element_type=jnp.float32) + bih
        gh = jnp.dot(h, whh, preferred_element_type=jnp.float32) + bhh
        r = jax.nn.sigmoid(gi[:, 0:16] + gh[:, 0:16])
        z = jax.nn.sigmoid(gi[:, 128:144] + gh[:, 128:144])
        n = jnp.tanh(gi[:, 256:272] + r * gh[:, 256:272])
        h = (1.0 - z) * n + z * h
    hh = jnp.maximum(
        jnp.dot(h, w1_ref[...], preferred_element_type=jnp.float32) + b1_ref[...],
        0.0)
    o = jnp.dot(hh, w2_ref[...], preferred_element_type=jnp.float32) + b2_ref[...]
    o_ref[...] = _ln(o, g_ref[...], bt_ref[...])


def _gru_genc(xt, p, row_block, interpret=False):
    n = xt.shape[1]
    wih = p['gru_Wih']
    whh = p['gru_Whh']
    bih = p['gru_bih']
    bhh = p['gru_bhh']
    wih_p = jnp.zeros((16, 384), jnp.float32)
    whh_p = jnp.zeros((16, 384), jnp.float32)
    bih_p = jnp.zeros((1, 384), jnp.float32)
    bhh_p = jnp.zeros((1, 384), jnp.float32)
    for k in range(3):
        wih_p = wih_p.at[:, 128 * k:128 * k + 10].set(wih[:, 10 * k:10 * k + 10])
        whh_p = whh_p.at[0:10, 128 * k:128 * k + 10].set(whh[:, 10 * k:10 * k + 10])
        bih_p = bih_p.at[:, 128 * k:128 * k + 10].set(bih[10 * k:10 * k + 10])
        bhh_p = bhh_p.at[:, 128 * k:128 * k + 10].set(bhh[10 * k:10 * k + 10])
    m = p['genc']
    w1_p = jnp.zeros((16, H), jnp.float32).at[0:10, :].set(m['W1'])
    ws = (w1_p, m['b1'].reshape(1, -1), m['W2'], m['b2'].reshape(1, -1),
          m['g'].reshape(1, -1), m['bt'].reshape(1, -1))
    args = (wih_p, bih_p, whh_p, bhh_p) + ws
    return pl.pallas_call(
        _gru_body,
        grid=(n // row_block,),
        in_specs=[pl.BlockSpec((8, row_block, 16), lambda i: (0, i, 0))]
        + [_wspec(w) for w in args],
        out_specs=pl.BlockSpec((row_block, H), lambda i: (i, 0)),
        out_shape=jax.ShapeDtypeStruct((n, H), jnp.float32),
        interpret=interpret,
    )(xt, *args)


def _edge_body(a_ref, b_ref, c_ref, w1a_ref, w1b_ref, w1c_ref, b1_ref,
               w2_ref, b2_ref, g_ref, bt_ref, o_ref):
    c = c_ref[...]
    h = (jnp.dot(a_ref[...], w1a_ref[...], preferred_element_type=jnp.float32)
         + jnp.dot(b_ref[...], w1b_ref[...], preferred_element_type=jnp.float32)
         + jnp.dot(c, w1c_ref[...], preferred_element_type=jnp.float32)
         + b1_ref[...])
    h = jnp.maximum(h, 0.0)
    o = jnp.dot(h, w2_ref[...], preferred_element_type=jnp.float32) + b2_ref[...]
    o_ref[...] = c + _ln(o, g_ref[...], bt_ref[...])


def _edge_mlp(a, b, c, m, row_block, interpret=False):
    n = a.shape[0]
    w1, b1, w2, b2, g, bt = _mw(m)
    args = (w1[0:H], w1[H:2 * H], w1[2 * H:3 * H], b1, w2, b2, g, bt)
    return pl.pallas_call(
        _edge_body,
        grid=(n // row_block,),
        in_specs=[pl.BlockSpec((row_block, H), lambda i: (i, 0))] * 3
        + [_wspec(w) for w in args],
        out_specs=pl.BlockSpec((row_block, H), lambda i: (i, 0)),
        out_shape=jax.ShapeDtypeStruct((n, H), jnp.float32),
        interpret=interpret,
    )(a, b, c, *args)


def _node_body(x_ref, a_ref, w1x_ref, w1a_ref, b1_ref, w2_ref, b2_ref,
               g_ref, bt_ref, o_ref):
    x = x_ref[...]
    a = a_ref[0]
    for k in range(1, a_ref.shape[0]):
        a = a + a_ref[k]
    h = (jnp.dot(x, w1x_ref[...], preferred_element_type=jnp.float32)
         + jnp.dot(a, w1a_ref[...], preferred_element_type=jnp.float32)
         + b1_ref[...])
    h = jnp.maximum(h, 0.0)
    o = jnp.dot(h, w2_ref[...], preferred_element_type=jnp.float32) + b2_ref[...]
    o_ref[...] = x + _ln(o, g_ref[...], bt_ref[...])


def _node_mlp(x, agg, m, row_block, interpret=False):
    n = x.shape[0]
    k = agg.shape[0]
    w1, b1, w2, b2, g, bt = _mw(m)
    args = (w1[0:H], w1[H:2 * H], b1, w2, b2, g, bt)
    return pl.pallas_call(
        _node_body,
        grid=(n // row_block,),
        in_specs=[pl.BlockSpec((row_block, H), lambda i: (i, 0)),
                  pl.BlockSpec((k, row_block, H), lambda i: (0, i, 0))]
        + [_wspec(w) for w in args],
        out_specs=pl.BlockSpec((row_block, H), lambda i: (i, 0)),
        out_shape=jax.ShapeDtypeStruct((n, H), jnp.float32),
        interpret=interpret,
    )(x, agg, *args)


def _head_body(x_ref, dw1_ref, db1_ref, dw2_ref, db2_ref, l1w_ref, l1b_ref,
               l2w_ref, l2b_ref, ow_ref, ob_ref, o_ref):
    h = jnp.maximum(
        jnp.dot(x_ref[...], dw1_ref[...], preferred_element_type=jnp.float32)
        + db1_ref[...], 0.0)
    dec = jnp.dot(h, dw2_ref[...], preferred_element_type=jnp.float32) + db2_ref[...]
    xx = jnp.maximum(dec * l1w_ref[...] + l1b_ref[...], 0.0)
    hid = jnp.maximum(
        jnp.dot(xx, l2w_ref[...], preferred_element_type=jnp.float32)
        + l2b_ref[...], 0.0)
    logits = jnp.dot(hid, ow_ref[...], preferred_element_type=jnp.float32) + ob_ref[...]
    mx = jnp.max(logits, axis=-1, keepdims=True)
    e = jnp.exp(logits - mx)
    o_ref[...] = e / jnp.sum(e, axis=-1, keepdims=True)


def _head(x, p, row_block, interpret=False):
    n = x.shape[0]
    args = (p['dec_W1'], p['dec_b1'].reshape(1, -1), p['dec_W2'],
            p['dec_b2'].reshape(1, -1), p['lin1_W'], p['lin1_b'].reshape(1, -1),
            p['lin2_W'], p['lin2_b'].reshape(1, -1), p['out_W'],
            p['out_b'].reshape(1, -1))
    return pl.pallas_call(
        _head_body,
        grid=(n // row_block,),
        in_specs=[pl.BlockSpec((row_block, H), lambda i: (i, 0))]
        + [_wspec(w) for w in args],
        out_specs=pl.BlockSpec((row_block, 4), lambda i: (i, 0)),
        out_shape=jax.ShapeDtypeStruct((n, 4), jnp.float32),
        interpret=interpret,
    )(x, *args)


def _pad_rows(a, n):
    return jnp.pad(a, ((0, n - a.shape[0]),) + ((0, 0),) * (a.ndim - 1))


def _gather(table, idx, n_valid):
    return jnp.take(table, idx, axis=0)


def _segsum(vals, dst, n_seg):
    return jax.ops.segment_sum(vals, dst, num_segments=n_seg)[None]


def kernel(X, mesh_x, graph_edge_attr, g2m_edge_attr, m2g_edge_attr,
           graph_edge_index, g2m_src, g2m_dst, m2g_src, m2g_dst, params):
    p = params
    n_grid = X.shape[0]
    n_mesh = mesh_x.shape[0]
    e_mesh = graph_edge_attr.shape[0]
    e_g2m = g2m_edge_attr.shape[0]
    e_m2g = m2g_edge_attr.shape[0]

    def pad_to(e):
        per = -(-e // (_NW * _CHUNK))
        return _NW * _CHUNK * per

    ep_mesh = pad_to(e_mesh)
    ep_g2m = pad_to(e_g2m)
    ep_m2g = pad_to(e_m2g)

    xt = jnp.transpose(X, (2, 0, 1))
    grid = _gru_genc(xt, p, 2000)
    mesh = _enc(mesh_x, p['menc'], 2000)
    me = _enc(_pad_rows(graph_edge_attr, ep_mesh), p['eenc'], 2048)
    g2me = _enc(_pad_rows(g2m_edge_attr, ep_g2m), p['g2menc'], 2048)
    m2ge = _enc(_pad_rows(m2g_edge_attr, ep_m2g), p['m2genc'], 2048)

    def pad_idx(idx, n, fill):
        return jnp.pad(idx, (0, n - idx.shape[0]), constant_values=fill)

    g2m_s = pad_idx(g2m_src, ep_g2m, 0)
    g2m_d = pad_idx(g2m_dst, ep_g2m, n_mesh)
    mm_s = pad_idx(graph_edge_index[0], ep_mesh, 0)
    mm_d = pad_idx(graph_edge_index[1], ep_mesh, n_mesh)
    m2g_s = pad_idx(m2g_src, ep_m2g, 0)
    m2g_d = pad_idx(m2g_dst, ep_m2g, n_grid)

    ga = _gather(grid, g2m_s, e_g2m)
    gb = _gather(mesh, jnp.where(g2m_d >= n_mesh, 0, g2m_d), e_g2m)
    g2me = _edge_mlp(ga, gb, g2me, p['g2m_e'], 2048)
    agg = _segsum(g2me, g2m_d, n_mesh)
    mesh = _node_mlp(mesh, agg, p['g2m_n'], 2000)

    mm_d_safe = jnp.where(mm_d >= n_mesh, 0, mm_d)
    for lp in p['proc']:
        ga = _gather(mesh, mm_s, e_mesh)
        gb = _gather(mesh, mm_d_safe, e_mesh)
        me = _edge_mlp(ga, gb, me, lp['e'], 2048)
        agg = _segsum(me, mm_d, n_mesh)
        mesh = _node_mlp(mesh, agg, lp['n'], 2000)

    ga = _gather(mesh, m2g_s, e_m2g)
    gb = _gather(grid, jnp.where(m2g_d >= n_grid, 0, m2g_d), e_m2g)
    m2ge = _edge_mlp(ga, gb, m2ge, p['m2g_e'], 2048)
    agg = _segsum(m2ge, m2g_d, n_grid)
    grid = _node_mlp(grid, agg, p['m2g_n'], 2000)

    return _head(grid, p, 2000)
